# Initial kernel scaffold; baseline (speedup 1.0000x reference)
import functools

import jax
import jax.numpy as jnp
from jax import lax
from jax.experimental import pallas as pl
from jax.experimental.pallas import tpu as pltpu
from jax.experimental.pallas import tpu_sc as plsc

N = 10000
E = 320000
D = 128
NC = 2
NS = 16
W = 80
EPW = E // (NC * NS)
NWIN = EPW // W
NCH = N // W
CW = 16


def _make_sc_agg(with_count: bool):
    mesh = plsc.VectorSubcoreMesh(core_axis_name="c", subcore_axis_name="s")
    out_type = [jax.ShapeDtypeStruct((NC, N, D), jnp.float32)]
    scratch = [
        pltpu.VMEM((W,), jnp.int32),
        pltpu.VMEM((W,), jnp.int32),
        pltpu.VMEM((W, D), jnp.float32),
        pltpu.VMEM_SHARED((N, D), jnp.float32),
        pltpu.SemaphoreType.DMA,
    ]
    if with_count:
        out_type.append(jax.ShapeDtypeStruct((NC, N, CW), jnp.float32))
        scratch += [
            pltpu.VMEM((W, CW), jnp.float32),
            pltpu.VMEM((W, CW), jnp.float32),
            pltpu.VMEM_SHARED((N, CW), jnp.float32),
        ]

    @functools.partial(pl.kernel, mesh=mesh, out_type=out_type,
                       scratch_types=scratch)
    def sc_agg(x_hbm, src_hbm, dst_hbm, sum_hbm, *rest):
        if with_count:
            (cnt_hbm, src_v, dst_v, rows_v, acc_sh, sem,
             ones_v, zc_v, cnt_sh) = rest
        else:
            src_v, dst_v, rows_v, acc_sh, sem = rest
        c = lax.axis_index("c")
        s = lax.axis_index("s")

        @pl.loop(0, W)
        def _fill(i):
            for j in range(D // 16):
                rows_v[i, pl.ds(j * 16, 16)] = jnp.zeros((16,), jnp.float32)
            if with_count:
                ones_v[i, pl.ds(0, 16)] = jnp.ones((16,), jnp.float32)
                zc_v[i, pl.ds(0, 16)] = jnp.zeros((16,), jnp.float32)

        @pl.loop(s, NCH, step=NS)
        def _zero(ch):
            pltpu.sync_copy(rows_v, acc_sh.at[pl.ds(ch * W, W)])
            if with_count:
                pltpu.sync_copy(zc_v, cnt_sh.at[pl.ds(ch * W, W)])

        plsc.subcore_barrier()

        base = (c * NS + s) * EPW

        @pl.loop(0, NWIN)
        def _edges(w):
            off = base + w * W
            pltpu.sync_copy(src_hbm.at[pl.ds(off, W)], src_v)
            pltpu.sync_copy(dst_hbm.at[pl.ds(off, W)], dst_v)
            pltpu.async_copy(x_hbm.at[src_v], rows_v, sem).wait()
            pltpu.sync_copy(rows_v, acc_sh.at[dst_v], add=True)
            if with_count:
                pltpu.sync_copy(ones_v, cnt_sh.at[dst_v], add=True)

        plsc.subcore_barrier()

        @pl.loop(s, NCH, step=NS)
        def _drain(ch):
            pltpu.sync_copy(acc_sh.at[pl.ds(ch * W, W)],
                            sum_hbm.at[c, pl.ds(ch * W, W)])
            if with_count:
                pltpu.sync_copy(cnt_sh.at[pl.ds(ch * W, W)],
                                cnt_hbm.at[c, pl.ds(ch * W, W)])

    return sc_agg


_sc_agg_count = _make_sc_agg(True)
_sc_agg_plain = _make_sc_agg(False)


def _dense(sums, cnts, x, WlT, bl2d, WrT, relu: bool):
    RB = 1250

    def body(s_ref, c_ref, x_ref, wl_ref, b_ref, wr_ref, o_ref):
        ssum = s_ref[0] + s_ref[1]
        cnt = c_ref[0, :, 0:1] + c_ref[1, :, 0:1]
        mean = ssum / jnp.maximum(cnt, 1.0)
        out = (jnp.dot(mean, wl_ref[...], preferred_element_type=jnp.float32)
               + jnp.dot(x_ref[...], wr_ref[...],
                         preferred_element_type=jnp.float32)
               + b_ref[...])
        nrm = jnp.sqrt(jnp.sum(out * out, axis=1, keepdims=True))
        out = out / jnp.maximum(nrm, 1e-12)
        if relu:
            out = jnp.maximum(out, 0.0)
        o_ref[...] = out

    return pl.pallas_call(
        body,
        grid=(N // RB,),
        in_specs=[
            pl.BlockSpec((NC, RB, D), lambda i: (0, i, 0)),
            pl.BlockSpec((NC, RB, CW), lambda i: (0, i, 0)),
            pl.BlockSpec((RB, D), lambda i: (i, 0)),
            pl.BlockSpec((D, D), lambda i: (0, 0)),
            pl.BlockSpec((1, D), lambda i: (0, 0)),
            pl.BlockSpec((D, D), lambda i: (0, 0)),
        ],
        out_specs=pl.BlockSpec((RB, D), lambda i: (i, 0)),
        out_shape=jax.ShapeDtypeStruct((N, D), jnp.float32),
    )(sums, cnts, x, WlT, bl2d, WrT)


def kernel(x, edge_index, Wl1, bl1, Wr1, Wl2, bl2, Wr2):
    src = edge_index[0].astype(jnp.int32)
    dst = edge_index[1].astype(jnp.int32)
    sums1, cnts = _sc_agg_count(x, src, dst)
    h = _dense(sums1, cnts, x, Wl1.T, bl1.reshape(1, D), Wr1.T, relu=True)
    sums2 = _sc_agg_plain(h, src, dst)
    if isinstance(sums2, (list, tuple)):
        sums2 = sums2[0]
    out = _dense(sums2, cnts, h, Wl2.T, bl2.reshape(1, D), Wr2.T, relu=False)
    return out


# speedup vs baseline: 2.6024x; 2.6024x over previous
import functools

import jax
import jax.numpy as jnp
from jax import lax
from jax.experimental import pallas as pl
from jax.experimental.pallas import tpu as pltpu
from jax.experimental.pallas import tpu_sc as plsc

N = 10000
NP = 10240
E = 320000
D = 128
NC = 1
NS = 16
W = 80
EPW = E // (NC * NS)
NWIN = EPW // W
RPS = NP // NS
ZR = 80
NZ = RPS // ZR
CW = 128

_MESH = plsc.VectorSubcoreMesh(core_axis_name="c", subcore_axis_name="s",
                               num_cores=NC)


def _fill_idx(idx_v, row0):
    for j in range(W // 16):
        idx_v[pl.ds(j * 16, 16)] = lax.iota(jnp.int32, 16) + (row0 + j * 16)


@functools.partial(
    pl.kernel, mesh=_MESH,
    out_type=jax.ShapeDtypeStruct((NP, D), jnp.float32),
    scratch_types=[
        pltpu.VMEM((W,), jnp.int32),
        pltpu.VMEM((W,), jnp.int32),
        pltpu.VMEM((W,), jnp.int32),
        pltpu.VMEM((W, D), jnp.float32),
        pltpu.VMEM_SHARED((NP, D), jnp.float32),
        pltpu.SemaphoreType.DMA,
    ])
def _sc_agg(x_hbm, src_hbm, dst_hbm, sum_hbm,
            src_v, dst_v, idx_v, rows_v, acc_sh, sem):
    s = lax.axis_index("s")

    @pl.loop(0, W)
    def _fill_z(i):
        for j in range(D // 16):
            rows_v[i, pl.ds(j * 16, 16)] = jnp.zeros((16,), jnp.float32)

    for k in range(NZ):
        row0 = s * RPS + k * ZR
        _fill_idx(idx_v, row0)
        pltpu.sync_copy(rows_v, acc_sh.at[idx_v])

    plsc.subcore_barrier()

    base = s * EPW

    @pl.loop(0, NWIN)
    def _edges(w):
        off = base + w * W
        pltpu.sync_copy(src_hbm.at[pl.ds(off, W)], src_v)
        pltpu.sync_copy(dst_hbm.at[pl.ds(off, W)], dst_v)
        pltpu.async_copy(x_hbm.at[src_v], rows_v, sem).wait()
        pltpu.sync_copy(rows_v, acc_sh.at[dst_v], add=True)

    plsc.subcore_barrier()

    for k in range(NZ):
        row0 = s * RPS + k * ZR
        _fill_idx(idx_v, row0)
        pltpu.async_copy(acc_sh.at[idx_v], rows_v, sem).wait()
        pltpu.sync_copy(rows_v, sum_hbm.at[pl.ds(row0, ZR)])


@functools.partial(
    pl.kernel, mesh=_MESH,
    out_type=jax.ShapeDtypeStruct((NP, CW), jnp.float32),
    scratch_types=[
        pltpu.VMEM((W,), jnp.int32),
        pltpu.VMEM((W,), jnp.int32),
        pltpu.VMEM((W, CW), jnp.float32),
        pltpu.VMEM((W, CW), jnp.float32),
        pltpu.VMEM_SHARED((NP, CW), jnp.float32),
        pltpu.SemaphoreType.DMA,
    ])
def _sc_cnt(dst_hbm, cnt_hbm, dst_v, idx_v, ones_v, zc_v, cnt_sh, sem):
    s = lax.axis_index("s")

    @pl.loop(0, W)
    def _fill(i):
        for j in range(CW // 16):
            ones_v[i, pl.ds(j * 16, 16)] = jnp.ones((16,), jnp.float32)
            zc_v[i, pl.ds(j * 16, 16)] = jnp.zeros((16,), jnp.float32)

    for k in range(NZ):
        row0 = s * RPS + k * ZR
        _fill_idx(idx_v, row0)
        pltpu.sync_copy(zc_v, cnt_sh.at[idx_v])

    plsc.subcore_barrier()

    base = s * EPW

    @pl.loop(0, NWIN)
    def _edges(w):
        pltpu.sync_copy(dst_hbm.at[pl.ds(base + w * W, W)], dst_v)
        pltpu.sync_copy(ones_v, cnt_sh.at[dst_v], add=True)

    plsc.subcore_barrier()

    for k in range(NZ):
        row0 = s * RPS + k * ZR
        _fill_idx(idx_v, row0)
        pltpu.async_copy(cnt_sh.at[idx_v], zc_v, sem).wait()
        pltpu.sync_copy(zc_v, cnt_hbm.at[pl.ds(row0, ZR)])


def _dense(s0, c0, x, WlT, bl2d, WrT, relu: bool):
    RB = 2000

    def body(s0_ref, c0_ref, x_ref, wl_ref, b_ref, wr_ref, o_ref):
        cnt = c0_ref[:, 0:1]
        mean = s0_ref[...] / jnp.maximum(cnt, 1.0)
        out = (jnp.dot(mean, wl_ref[...], preferred_element_type=jnp.float32)
               + jnp.dot(x_ref[...], wr_ref[...],
                         preferred_element_type=jnp.float32)
               + b_ref[...])
        nrm = jnp.sqrt(jnp.sum(out * out, axis=1, keepdims=True))
        out = out / jnp.maximum(nrm, 1e-12)
        if relu:
            out = jnp.maximum(out, 0.0)
        o_ref[...] = out

    return pl.pallas_call(
        body,
        grid=(N // RB,),
        in_specs=[
            pl.BlockSpec((RB, D), lambda i: (i, 0)),
            pl.BlockSpec((RB, CW), lambda i: (i, 0)),
            pl.BlockSpec((RB, D), lambda i: (i, 0)),
            pl.BlockSpec((D, D), lambda i: (0, 0)),
            pl.BlockSpec((1, D), lambda i: (0, 0)),
            pl.BlockSpec((D, D), lambda i: (0, 0)),
        ],
        out_specs=pl.BlockSpec((RB, D), lambda i: (i, 0)),
        out_shape=jax.ShapeDtypeStruct((N, D), jnp.float32),
    )(s0, c0, x, WlT, bl2d, WrT)


def kernel(x, edge_index, Wl1, bl1, Wr1, Wl2, bl2, Wr2):
    src = edge_index[0].astype(jnp.int32)
    dst = edge_index[1].astype(jnp.int32)
    cnts = _sc_cnt(dst)
    sums1 = _sc_agg(x, src, dst)
    h = _dense(sums1[:N], cnts[:N], x, Wl1.T, bl1.reshape(1, D), Wr1.T,
               relu=True)
    sums2 = _sc_agg(h, src, dst)
    out = _dense(sums2[:N], cnts[:N], h, Wl2.T, bl2.reshape(1, D), Wr2.T,
                 relu=False)
    return out


# speedup vs baseline: 4.7508x; 1.8255x over previous
import functools

import jax
import jax.numpy as jnp
from jax import lax
from jax.experimental import pallas as pl
from jax.experimental.pallas import tpu as pltpu
from jax.experimental.pallas import tpu_sc as plsc

N = 10000
NP = 10240
E = 320000
D = 128
NC = 2
NS = 16
W = 80
EPW = E // (NC * NS)
NWIN = EPW // W
RPS = NP // NS
ZR = 80
NZ = RPS // ZR
CW = 128

_MESH = plsc.VectorSubcoreMesh(core_axis_name="c", subcore_axis_name="s",
                               num_cores=NC)


def _fill_idx(idx_v, row0):
    for j in range(W // 16):
        idx_v[pl.ds(j * 16, 16)] = lax.iota(jnp.int32, 16) + (row0 + j * 16)


@functools.partial(
    pl.kernel, mesh=_MESH,
    out_type=jax.ShapeDtypeStruct((NC * NP, D), jnp.float32),
    scratch_types=[
        pltpu.VMEM((W,), jnp.int32),
        pltpu.VMEM((W,), jnp.int32),
        pltpu.VMEM((W,), jnp.int32),
        pltpu.VMEM((W, D), jnp.float32),
        pltpu.VMEM_SHARED((NP, D), jnp.float32),
        pltpu.SemaphoreType.DMA,
    ])
def _sc_agg(x_hbm, src_hbm, dst_hbm, sum_hbm,
            src_v, dst_v, idx_v, rows_v, acc_sh, sem):
    c = lax.axis_index("c")
    s = lax.axis_index("s")

    @pl.loop(0, W)
    def _fill_z(i):
        for j in range(D // 16):
            rows_v[i, pl.ds(j * 16, 16)] = jnp.zeros((16,), jnp.float32)

    for k in range(NZ):
        row0 = s * RPS + k * ZR
        _fill_idx(idx_v, row0)
        pltpu.sync_copy(rows_v, acc_sh.at[idx_v])

    plsc.subcore_barrier()

    base = (c * NS + s) * EPW

    @pl.loop(0, NWIN)
    def _edges(w):
        off = base + w * W
        pltpu.sync_copy(src_hbm.at[pl.ds(off, W)], src_v)
        pltpu.sync_copy(dst_hbm.at[pl.ds(off, W)], dst_v)
        pltpu.async_copy(x_hbm.at[src_v], rows_v, sem).wait()
        pltpu.sync_copy(rows_v, acc_sh.at[dst_v], add=True)

    plsc.subcore_barrier()

    for k in range(NZ):
        row0 = s * RPS + k * ZR
        _fill_idx(idx_v, row0)
        pltpu.async_copy(acc_sh.at[idx_v], rows_v, sem).wait()
        pltpu.sync_copy(rows_v, sum_hbm.at[pl.ds(c * NP + row0, ZR)])


@functools.partial(
    pl.kernel, mesh=_MESH,
    out_type=jax.ShapeDtypeStruct((NC * NP, CW), jnp.float32),
    scratch_types=[
        pltpu.VMEM((W,), jnp.int32),
        pltpu.VMEM((W,), jnp.int32),
        pltpu.VMEM((W, CW), jnp.float32),
        pltpu.VMEM((W, CW), jnp.float32),
        pltpu.VMEM_SHARED((NP, CW), jnp.float32),
        pltpu.SemaphoreType.DMA,
    ])
def _sc_cnt(dst_hbm, cnt_hbm, dst_v, idx_v, ones_v, zc_v, cnt_sh, sem):
    c = lax.axis_index("c")
    s = lax.axis_index("s")

    @pl.loop(0, W)
    def _fill(i):
        for j in range(CW // 16):
            ones_v[i, pl.ds(j * 16, 16)] = jnp.ones((16,), jnp.float32)
            zc_v[i, pl.ds(j * 16, 16)] = jnp.zeros((16,), jnp.float32)

    for k in range(NZ):
        row0 = s * RPS + k * ZR
        _fill_idx(idx_v, row0)
        pltpu.sync_copy(zc_v, cnt_sh.at[idx_v])

    plsc.subcore_barrier()

    base = (c * NS + s) * EPW

    @pl.loop(0, NWIN)
    def _edges(w):
        pltpu.sync_copy(dst_hbm.at[pl.ds(base + w * W, W)], dst_v)
        pltpu.sync_copy(ones_v, cnt_sh.at[dst_v], add=True)

    plsc.subcore_barrier()

    for k in range(NZ):
        row0 = s * RPS + k * ZR
        _fill_idx(idx_v, row0)
        pltpu.async_copy(cnt_sh.at[idx_v], zc_v, sem).wait()
        pltpu.sync_copy(zc_v, cnt_hbm.at[pl.ds(c * NP + row0, ZR)])


def _dense(s0, s1, c0, c1, x, WlT, bl2d, WrT, relu: bool):
    RB = 2000

    def body(s0_ref, s1_ref, c0_ref, c1_ref, x_ref, wl_ref, b_ref, wr_ref,
             o_ref):
        cnt = c0_ref[:, 0:1] + c1_ref[:, 0:1]
        mean = (s0_ref[...] + s1_ref[...]) / jnp.maximum(cnt, 1.0)
        out = (jnp.dot(mean, wl_ref[...], preferred_element_type=jnp.float32)
               + jnp.dot(x_ref[...], wr_ref[...],
                         preferred_element_type=jnp.float32)
               + b_ref[...])
        nrm = jnp.sqrt(jnp.sum(out * out, axis=1, keepdims=True))
        out = out / jnp.maximum(nrm, 1e-12)
        if relu:
            out = jnp.maximum(out, 0.0)
        o_ref[...] = out

    return pl.pallas_call(
        body,
        grid=(N // RB,),
        in_specs=[
            pl.BlockSpec((RB, D), lambda i: (i, 0)),
            pl.BlockSpec((RB, D), lambda i: (i, 0)),
            pl.BlockSpec((RB, CW), lambda i: (i, 0)),
            pl.BlockSpec((RB, CW), lambda i: (i, 0)),
            pl.BlockSpec((RB, D), lambda i: (i, 0)),
            pl.BlockSpec((D, D), lambda i: (0, 0)),
            pl.BlockSpec((1, D), lambda i: (0, 0)),
            pl.BlockSpec((D, D), lambda i: (0, 0)),
        ],
        out_specs=pl.BlockSpec((RB, D), lambda i: (i, 0)),
        out_shape=jax.ShapeDtypeStruct((N, D), jnp.float32),
    )(s0, s1, c0, c1, x, WlT, bl2d, WrT)


def kernel(x, edge_index, Wl1, bl1, Wr1, Wl2, bl2, Wr2):
    src = edge_index[0].astype(jnp.int32)
    dst = edge_index[1].astype(jnp.int32)
    cnts = _sc_cnt(dst)
    c0, c1 = cnts[:N], cnts[NP:NP + N]
    sums1 = _sc_agg(x, src, dst)
    h = _dense(sums1[:N], sums1[NP:NP + N], c0, c1, x, Wl1.T,
               bl1.reshape(1, D), Wr1.T, relu=True)
    sums2 = _sc_agg(h, src, dst)
    out = _dense(sums2[:N], sums2[NP:NP + N], c0, c1, h, Wl2.T,
                 bl2.reshape(1, D), Wr2.T, relu=False)
    return out


# speedup vs baseline: 8.5508x; 1.7999x over previous
import functools

import jax
import jax.numpy as jnp
from jax import lax
from jax.experimental import pallas as pl
from jax.experimental.pallas import tpu as pltpu
from jax.experimental.pallas import tpu_sc as plsc

N = 10000
NP = 10240
E = 320000
D = 128
NC = 2
NS = 16
W = 80
EPW = E // (NC * NS)
NWIN = EPW // W
RPS = NP // NS
ZR = 80
NZ = RPS // ZR
CW = 128

_MESH = plsc.VectorSubcoreMesh(core_axis_name="c", subcore_axis_name="s",
                               num_cores=NC)


def _fill_idx(idx_v, row0):
    for j in range(W // 16):
        idx_v[pl.ds(j * 16, 16)] = lax.iota(jnp.int32, 16) + (row0 + j * 16)


IBW = 25
EB = IBW * W
NB = EPW // EB


@functools.partial(
    pl.kernel, mesh=_MESH,
    out_type=jax.ShapeDtypeStruct((NC * NP, D), jnp.float32),
    scratch_types=[
        pltpu.VMEM((EB,), jnp.int32),
        pltpu.VMEM((EB,), jnp.int32),
        pltpu.VMEM((W,), jnp.int32),
        pltpu.VMEM((W,), jnp.int32),
        pltpu.VMEM((W,), jnp.int32),
        pltpu.VMEM((W, D), jnp.float32),
        pltpu.VMEM((W, D), jnp.float32),
        pltpu.VMEM_SHARED((NP, D), jnp.float32),
        pltpu.SemaphoreType.DMA,
        pltpu.SemaphoreType.DMA,
    ])
def _sc_agg(x_hbm, src_hbm, dst_hbm, sum_hbm,
            src_b, dst_b, dstv0, dstv1, idx_v, rows0, rows1, acc_sh,
            sem0, sem1):
    c = lax.axis_index("c")
    s = lax.axis_index("s")
    dstv = (dstv0, dstv1)
    rows = (rows0, rows1)
    sems = (sem0, sem1)

    @pl.loop(0, W)
    def _fill_z(i):
        for j in range(D // 16):
            rows0[i, pl.ds(j * 16, 16)] = jnp.zeros((16,), jnp.float32)

    for k in range(NZ):
        row0 = s * RPS + k * ZR
        _fill_idx(idx_v, row0)
        pltpu.sync_copy(rows0, acc_sh.at[idx_v])

    plsc.subcore_barrier()

    base = (c * NS + s) * EPW

    @pl.loop(0, NB)
    def _block(b):
        off = base + b * EB
        pltpu.sync_copy(src_hbm.at[pl.ds(off, EB)], src_b)
        pltpu.sync_copy(dst_hbm.at[pl.ds(off, EB)], dst_b)
        handles = [None, None]
        handles[0] = pltpu.async_copy(
            x_hbm.at[src_b.at[pl.ds(0, W)]], rows[0], sems[0])
        for j in range(IBW):
            p = j % 2
            if j + 1 < IBW:
                q = (j + 1) % 2
                handles[q] = pltpu.async_copy(
                    x_hbm.at[src_b.at[pl.ds((j + 1) * W, W)]], rows[q],
                    sems[q])
            handles[p].wait()
            pltpu.sync_copy(rows[p], acc_sh.at[dst_b.at[pl.ds(j * W, W)]],
                            add=True)

    plsc.subcore_barrier()

    for k in range(NZ):
        row0 = s * RPS + k * ZR
        _fill_idx(idx_v, row0)
        pltpu.async_copy(acc_sh.at[idx_v], rows0, sem0).wait()
        pltpu.sync_copy(rows0, sum_hbm.at[pl.ds(c * NP + row0, ZR)])


@functools.partial(
    pl.kernel, mesh=_MESH,
    out_type=jax.ShapeDtypeStruct((NC * NP, CW), jnp.float32),
    scratch_types=[
        pltpu.VMEM((W,), jnp.int32),
        pltpu.VMEM((W,), jnp.int32),
        pltpu.VMEM((W, CW), jnp.float32),
        pltpu.VMEM((W, CW), jnp.float32),
        pltpu.VMEM_SHARED((NP, CW), jnp.float32),
        pltpu.SemaphoreType.DMA,
    ])
def _sc_cnt(dst_hbm, cnt_hbm, dst_v, idx_v, ones_v, zc_v, cnt_sh, sem):
    c = lax.axis_index("c")
    s = lax.axis_index("s")

    @pl.loop(0, W)
    def _fill(i):
        for j in range(CW // 16):
            ones_v[i, pl.ds(j * 16, 16)] = jnp.ones((16,), jnp.float32)
            zc_v[i, pl.ds(j * 16, 16)] = jnp.zeros((16,), jnp.float32)

    for k in range(NZ):
        row0 = s * RPS + k * ZR
        _fill_idx(idx_v, row0)
        pltpu.sync_copy(zc_v, cnt_sh.at[idx_v])

    plsc.subcore_barrier()

    base = (c * NS + s) * EPW

    @pl.loop(0, NWIN)
    def _edges(w):
        pltpu.sync_copy(dst_hbm.at[pl.ds(base + w * W, W)], dst_v)
        pltpu.sync_copy(ones_v, cnt_sh.at[dst_v], add=True)

    plsc.subcore_barrier()

    for k in range(NZ):
        row0 = s * RPS + k * ZR
        _fill_idx(idx_v, row0)
        pltpu.async_copy(cnt_sh.at[idx_v], zc_v, sem).wait()
        pltpu.sync_copy(zc_v, cnt_hbm.at[pl.ds(c * NP + row0, ZR)])


def _dense(s0, s1, c0, c1, x, WlT, bl2d, WrT, relu: bool):
    RB = 2000

    def body(s0_ref, s1_ref, c0_ref, c1_ref, x_ref, wl_ref, b_ref, wr_ref,
             o_ref):
        cnt = c0_ref[:, 0:1] + c1_ref[:, 0:1]
        mean = (s0_ref[...] + s1_ref[...]) / jnp.maximum(cnt, 1.0)
        out = (jnp.dot(mean, wl_ref[...], preferred_element_type=jnp.float32)
               + jnp.dot(x_ref[...], wr_ref[...],
                         preferred_element_type=jnp.float32)
               + b_ref[...])
        nrm = jnp.sqrt(jnp.sum(out * out, axis=1, keepdims=True))
        out = out / jnp.maximum(nrm, 1e-12)
        if relu:
            out = jnp.maximum(out, 0.0)
        o_ref[...] = out

    return pl.pallas_call(
        body,
        grid=(N // RB,),
        in_specs=[
            pl.BlockSpec((RB, D), lambda i: (i, 0)),
            pl.BlockSpec((RB, D), lambda i: (i, 0)),
            pl.BlockSpec((RB, CW), lambda i: (i, 0)),
            pl.BlockSpec((RB, CW), lambda i: (i, 0)),
            pl.BlockSpec((RB, D), lambda i: (i, 0)),
            pl.BlockSpec((D, D), lambda i: (0, 0)),
            pl.BlockSpec((1, D), lambda i: (0, 0)),
            pl.BlockSpec((D, D), lambda i: (0, 0)),
        ],
        out_specs=pl.BlockSpec((RB, D), lambda i: (i, 0)),
        out_shape=jax.ShapeDtypeStruct((N, D), jnp.float32),
    )(s0, s1, c0, c1, x, WlT, bl2d, WrT)


def kernel(x, edge_index, Wl1, bl1, Wr1, Wl2, bl2, Wr2):
    src = edge_index[0].astype(jnp.int32)
    dst = edge_index[1].astype(jnp.int32)
    cnts = _sc_cnt(dst)
    c0, c1 = cnts[:N], cnts[NP:NP + N]
    sums1 = _sc_agg(x, src, dst)
    h = _dense(sums1[:N], sums1[NP:NP + N], c0, c1, x, Wl1.T,
               bl1.reshape(1, D), Wr1.T, relu=True)
    sums2 = _sc_agg(h, src, dst)
    out = _dense(sums2[:N], sums2[NP:NP + N], c0, c1, h, Wl2.T,
                 bl2.reshape(1, D), Wr2.T, relu=False)
    return out


# speedup vs baseline: 9.7802x; 1.1438x over previous
import functools

import jax
import jax.numpy as jnp
from jax import lax
from jax.experimental import pallas as pl
from jax.experimental.pallas import tpu as pltpu
from jax.experimental.pallas import tpu_sc as plsc

N = 10000
NP = 10240
E = 320000
D = 128
NC = 2
NS = 16
W = 80
EPW = E // (NC * NS)
NWIN = EPW // W
RPS = NP // NS
ZR = 80
NZ = RPS // ZR
CW = 128

_MESH = plsc.VectorSubcoreMesh(core_axis_name="c", subcore_axis_name="s",
                               num_cores=NC)


def _fill_idx(idx_v, row0, n=W):
    for j in range(n // 16):
        idx_v[pl.ds(j * 16, 16)] = lax.iota(jnp.int32, 16) + (row0 + j * 16)


IBW = 25
EB = IBW * W
NB = EPW // EB


@functools.partial(
    pl.kernel, mesh=_MESH,
    out_type=jax.ShapeDtypeStruct((NC * NP, D), jnp.float32),
    scratch_types=[
        pltpu.VMEM((EB,), jnp.int32),
        pltpu.VMEM((EB,), jnp.int32),
        pltpu.VMEM((RPS,), jnp.int32),
        pltpu.VMEM((W, D), jnp.float32),
        pltpu.VMEM((W, D), jnp.float32),
        pltpu.VMEM_SHARED((NP, D), jnp.float32),
        pltpu.SemaphoreType.DMA,
        pltpu.SemaphoreType.DMA,
    ])
def _sc_agg(x_hbm, src_hbm, dst_hbm, sum_hbm,
            src_b, dst_b, idx_v, rows0, rows1, acc_sh,
            sem0, sem1):
    c = lax.axis_index("c")
    s = lax.axis_index("s")
    rows = (rows0, rows1)
    sems = (sem0, sem1)

    @pl.loop(0, W)
    def _fill_z(i):
        for j in range(D // 16):
            rows0[i, pl.ds(j * 16, 16)] = jnp.zeros((16,), jnp.float32)

    _fill_idx(idx_v, s * RPS, RPS)

    zh = [pltpu.async_copy(rows0, acc_sh.at[idx_v.at[pl.ds(k * ZR, ZR)]],
                           sem0) for k in range(NZ)]
    for h in zh:
        h.wait()

    plsc.subcore_barrier()

    base = (c * NS + s) * EPW

    @pl.loop(0, NB)
    def _block(b):
        off = base + b * EB
        pltpu.sync_copy(src_hbm.at[pl.ds(off, EB)], src_b)
        pltpu.sync_copy(dst_hbm.at[pl.ds(off, EB)], dst_b)
        handles = [None, None]
        handles[0] = pltpu.async_copy(
            x_hbm.at[src_b.at[pl.ds(0, W)]], rows[0], sems[0])
        for j in range(IBW):
            p = j % 2
            if j + 1 < IBW:
                q = (j + 1) % 2
                handles[q] = pltpu.async_copy(
                    x_hbm.at[src_b.at[pl.ds((j + 1) * W, W)]], rows[q],
                    sems[q])
            handles[p].wait()
            pltpu.sync_copy(rows[p], acc_sh.at[dst_b.at[pl.ds(j * W, W)]],
                            add=True)

    plsc.subcore_barrier()

    dh = [None, None]
    dh[0] = pltpu.async_copy(acc_sh.at[idx_v.at[pl.ds(0, ZR)]], rows0, sem0)
    for k in range(NZ):
        p = k % 2
        if k + 1 < NZ:
            q = (k + 1) % 2
            dh[q] = pltpu.async_copy(
                acc_sh.at[idx_v.at[pl.ds((k + 1) * ZR, ZR)]], rows[q],
                sems[q])
        dh[p].wait()
        pltpu.sync_copy(rows[p],
                        sum_hbm.at[pl.ds(c * NP + s * RPS + k * ZR, ZR)])


@functools.partial(
    pl.kernel, mesh=_MESH,
    out_type=jax.ShapeDtypeStruct((NC * NP, CW), jnp.float32),
    scratch_types=[
        pltpu.VMEM((EB,), jnp.int32),
        pltpu.VMEM((RPS,), jnp.int32),
        pltpu.VMEM((W, CW), jnp.float32),
        pltpu.VMEM((W, CW), jnp.float32),
        pltpu.VMEM((W, CW), jnp.float32),
        pltpu.VMEM_SHARED((NP, CW), jnp.float32),
        pltpu.SemaphoreType.DMA,
        pltpu.SemaphoreType.DMA,
    ])
def _sc_cnt(dst_hbm, cnt_hbm, dst_b, idx_v, ones_v, zc0, zc1, cnt_sh,
            sem0, sem1):
    c = lax.axis_index("c")
    s = lax.axis_index("s")
    zc = (zc0, zc1)
    sems = (sem0, sem1)

    @pl.loop(0, W)
    def _fill(i):
        for j in range(CW // 16):
            ones_v[i, pl.ds(j * 16, 16)] = jnp.ones((16,), jnp.float32)
            zc0[i, pl.ds(j * 16, 16)] = jnp.zeros((16,), jnp.float32)

    _fill_idx(idx_v, s * RPS, RPS)

    zh = [pltpu.async_copy(zc0, cnt_sh.at[idx_v.at[pl.ds(k * ZR, ZR)]],
                           sem0) for k in range(NZ)]
    for h in zh:
        h.wait()

    plsc.subcore_barrier()

    base = (c * NS + s) * EPW

    @pl.loop(0, NB)
    def _block(b):
        pltpu.sync_copy(dst_hbm.at[pl.ds(base + b * EB, EB)], dst_b)
        for j in range(IBW):
            pltpu.sync_copy(ones_v, cnt_sh.at[dst_b.at[pl.ds(j * W, W)]],
                            add=True)

    plsc.subcore_barrier()

    dh = [None, None]
    dh[0] = pltpu.async_copy(cnt_sh.at[idx_v.at[pl.ds(0, ZR)]], zc0, sem0)
    for k in range(NZ):
        p = k % 2
        if k + 1 < NZ:
            q = (k + 1) % 2
            dh[q] = pltpu.async_copy(
                cnt_sh.at[idx_v.at[pl.ds((k + 1) * ZR, ZR)]], zc[q], sems[q])
        dh[p].wait()
        pltpu.sync_copy(zc[p],
                        cnt_hbm.at[pl.ds(c * NP + s * RPS + k * ZR, ZR)])


def _dense(s0, s1, c0, c1, x, WlT, bl2d, WrT, relu: bool):
    RB = 2000

    def body(s0_ref, s1_ref, c0_ref, c1_ref, x_ref, wl_ref, b_ref, wr_ref,
             o_ref):
        cnt = c0_ref[:, 0:1] + c1_ref[:, 0:1]
        mean = (s0_ref[...] + s1_ref[...]) / jnp.maximum(cnt, 1.0)
        out = (jnp.dot(mean, wl_ref[...], preferred_element_type=jnp.float32)
               + jnp.dot(x_ref[...], wr_ref[...],
                         preferred_element_type=jnp.float32)
               + b_ref[...])
        nrm = jnp.sqrt(jnp.sum(out * out, axis=1, keepdims=True))
        out = out / jnp.maximum(nrm, 1e-12)
        if relu:
            out = jnp.maximum(out, 0.0)
        o_ref[...] = out

    return pl.pallas_call(
        body,
        grid=(N // RB,),
        in_specs=[
            pl.BlockSpec((RB, D), lambda i: (i, 0)),
            pl.BlockSpec((RB, D), lambda i: (i, 0)),
            pl.BlockSpec((RB, CW), lambda i: (i, 0)),
            pl.BlockSpec((RB, CW), lambda i: (i, 0)),
            pl.BlockSpec((RB, D), lambda i: (i, 0)),
            pl.BlockSpec((D, D), lambda i: (0, 0)),
            pl.BlockSpec((1, D), lambda i: (0, 0)),
            pl.BlockSpec((D, D), lambda i: (0, 0)),
        ],
        out_specs=pl.BlockSpec((RB, D), lambda i: (i, 0)),
        out_shape=jax.ShapeDtypeStruct((N, D), jnp.float32),
    )(s0, s1, c0, c1, x, WlT, bl2d, WrT)


def kernel(x, edge_index, Wl1, bl1, Wr1, Wl2, bl2, Wr2):
    src = edge_index[0].astype(jnp.int32)
    dst = edge_index[1].astype(jnp.int32)
    cnts = _sc_cnt(dst)
    c0, c1 = cnts[:N], cnts[NP:NP + N]
    sums1 = _sc_agg(x, src, dst)
    h = _dense(sums1[:N], sums1[NP:NP + N], c0, c1, x, Wl1.T,
               bl1.reshape(1, D), Wr1.T, relu=True)
    sums2 = _sc_agg(h, src, dst)
    out = _dense(sums2[:N], sums2[NP:NP + N], c0, c1, h, Wl2.T,
                 bl2.reshape(1, D), Wr2.T, relu=False)
    return out


# speedup vs baseline: 10.0275x; 1.0253x over previous
import functools

import jax
import jax.numpy as jnp
from jax import lax
from jax.experimental import pallas as pl
from jax.experimental.pallas import tpu as pltpu
from jax.experimental.pallas import tpu_sc as plsc

N = 10000
NP = 10240
E = 320000
D = 128
NC = 2
NS = 16
W = 80
EPW = E // (NC * NS)
NWIN = EPW // W
RPS = NP // NS
ZR = 80
NZ = RPS // ZR
CW = 128

_MESH = plsc.VectorSubcoreMesh(core_axis_name="c", subcore_axis_name="s",
                               num_cores=NC)


def _fill_idx(idx_v, row0, n=W):
    for j in range(n // 16):
        idx_v[pl.ds(j * 16, 16)] = lax.iota(jnp.int32, 16) + (row0 + j * 16)


IBW = 25
EB = IBW * W
NB = EPW // EB


@functools.partial(
    pl.kernel, mesh=_MESH,
    out_type=jax.ShapeDtypeStruct((NC * NP, D), jnp.float32),
    scratch_types=[
        pltpu.VMEM((EB,), jnp.int32),
        pltpu.VMEM((EB,), jnp.int32),
        pltpu.VMEM((EB,), jnp.int32),
        pltpu.VMEM((EB,), jnp.int32),
        pltpu.VMEM((RPS,), jnp.int32),
        pltpu.VMEM((W, D), jnp.float32),
        pltpu.VMEM((W, D), jnp.float32),
        pltpu.VMEM_SHARED((NP, D), jnp.float32),
        pltpu.SemaphoreType.DMA,
        pltpu.SemaphoreType.DMA,
        pltpu.SemaphoreType.DMA,
    ])
def _sc_agg(x_hbm, src_hbm, dst_hbm, sum_hbm,
            src_b0, dst_b0, src_b1, dst_b1, idx_v, rows0, rows1, acc_sh,
            sem0, sem1, semi):
    c = lax.axis_index("c")
    s = lax.axis_index("s")
    rows = (rows0, rows1)
    sems = (sem0, sem1)

    @pl.loop(0, W)
    def _fill_z(i):
        for j in range(D // 16):
            rows0[i, pl.ds(j * 16, 16)] = jnp.zeros((16,), jnp.float32)

    _fill_idx(idx_v, s * RPS, RPS)

    zh = [pltpu.async_copy(rows0, acc_sh.at[idx_v.at[pl.ds(k * ZR, ZR)]],
                           sem0) for k in range(NZ)]
    for h in zh:
        h.wait()

    plsc.subcore_barrier()

    base = (c * NS + s) * EPW
    srcb = (src_b0, src_b1)
    dstb = (dst_b0, dst_b1)

    pltpu.sync_copy(src_hbm.at[pl.ds(base, EB)], src_b0)
    pltpu.sync_copy(dst_hbm.at[pl.ds(base, EB)], dst_b0)
    for b in range(NB):
        ib = b % 2
        bh = []
        if b + 1 < NB:
            off = base + (b + 1) * EB
            bh = [pltpu.async_copy(src_hbm.at[pl.ds(off, EB)],
                                   srcb[1 - ib], semi),
                  pltpu.async_copy(dst_hbm.at[pl.ds(off, EB)],
                                   dstb[1 - ib], semi)]
        handles = [None, None]
        handles[0] = pltpu.async_copy(
            x_hbm.at[srcb[ib].at[pl.ds(0, W)]], rows[0], sems[0])
        for j in range(IBW):
            p = j % 2
            if j + 1 < IBW:
                q = (j + 1) % 2
                handles[q] = pltpu.async_copy(
                    x_hbm.at[srcb[ib].at[pl.ds((j + 1) * W, W)]], rows[q],
                    sems[q])
            handles[p].wait()
            pltpu.sync_copy(rows[p],
                            acc_sh.at[dstb[ib].at[pl.ds(j * W, W)]],
                            add=True)
        for h in bh:
            h.wait()

    plsc.subcore_barrier()

    dh = [None, None]
    dh[0] = pltpu.async_copy(acc_sh.at[idx_v.at[pl.ds(0, ZR)]], rows0, sem0)
    for k in range(NZ):
        p = k % 2
        if k + 1 < NZ:
            q = (k + 1) % 2
            dh[q] = pltpu.async_copy(
                acc_sh.at[idx_v.at[pl.ds((k + 1) * ZR, ZR)]], rows[q],
                sems[q])
        dh[p].wait()
        pltpu.sync_copy(rows[p],
                        sum_hbm.at[pl.ds(c * NP + s * RPS + k * ZR, ZR)])


@functools.partial(
    pl.kernel, mesh=_MESH,
    out_type=jax.ShapeDtypeStruct((NC * NP, CW), jnp.float32),
    scratch_types=[
        pltpu.VMEM((EB,), jnp.int32),
        pltpu.VMEM((EB,), jnp.int32),
        pltpu.VMEM((RPS,), jnp.int32),
        pltpu.VMEM((W, CW), jnp.float32),
        pltpu.VMEM((W, CW), jnp.float32),
        pltpu.VMEM((W, CW), jnp.float32),
        pltpu.VMEM_SHARED((NP, CW), jnp.float32),
        pltpu.SemaphoreType.DMA,
        pltpu.SemaphoreType.DMA,
        pltpu.SemaphoreType.DMA,
    ])
def _sc_cnt(dst_hbm, cnt_hbm, dst_b0, dst_b1, idx_v, ones_v, zc0, zc1,
            cnt_sh, sem0, sem1, semi):
    c = lax.axis_index("c")
    s = lax.axis_index("s")
    zc = (zc0, zc1)
    sems = (sem0, sem1)

    @pl.loop(0, W)
    def _fill(i):
        for j in range(CW // 16):
            ones_v[i, pl.ds(j * 16, 16)] = jnp.ones((16,), jnp.float32)
            zc0[i, pl.ds(j * 16, 16)] = jnp.zeros((16,), jnp.float32)

    _fill_idx(idx_v, s * RPS, RPS)

    zh = [pltpu.async_copy(zc0, cnt_sh.at[idx_v.at[pl.ds(k * ZR, ZR)]],
                           sem0) for k in range(NZ)]
    for h in zh:
        h.wait()

    plsc.subcore_barrier()

    base = (c * NS + s) * EPW
    dstb = (dst_b0, dst_b1)

    pltpu.sync_copy(dst_hbm.at[pl.ds(base, EB)], dst_b0)
    for b in range(NB):
        ib = b % 2
        bh = []
        if b + 1 < NB:
            bh = [pltpu.async_copy(
                dst_hbm.at[pl.ds(base + (b + 1) * EB, EB)],
                dstb[1 - ib], semi)]
        for j in range(IBW):
            pltpu.sync_copy(ones_v,
                            cnt_sh.at[dstb[ib].at[pl.ds(j * W, W)]],
                            add=True)
        for h in bh:
            h.wait()

    plsc.subcore_barrier()

    dh = [None, None]
    dh[0] = pltpu.async_copy(cnt_sh.at[idx_v.at[pl.ds(0, ZR)]], zc0, sem0)
    for k in range(NZ):
        p = k % 2
        if k + 1 < NZ:
            q = (k + 1) % 2
            dh[q] = pltpu.async_copy(
                cnt_sh.at[idx_v.at[pl.ds((k + 1) * ZR, ZR)]], zc[q], sems[q])
        dh[p].wait()
        pltpu.sync_copy(zc[p],
                        cnt_hbm.at[pl.ds(c * NP + s * RPS + k * ZR, ZR)])


def _dense(s0, s1, c0, c1, x, WlT, bl2d, WrT, relu: bool):
    RB = 2000

    def body(s0_ref, s1_ref, c0_ref, c1_ref, x_ref, wl_ref, b_ref, wr_ref,
             o_ref):
        cnt = c0_ref[:, 0:1] + c1_ref[:, 0:1]
        mean = (s0_ref[...] + s1_ref[...]) / jnp.maximum(cnt, 1.0)
        out = (jnp.dot(mean, wl_ref[...], preferred_element_type=jnp.float32)
               + jnp.dot(x_ref[...], wr_ref[...],
                         preferred_element_type=jnp.float32)
               + b_ref[...])
        nrm = jnp.sqrt(jnp.sum(out * out, axis=1, keepdims=True))
        out = out / jnp.maximum(nrm, 1e-12)
        if relu:
            out = jnp.maximum(out, 0.0)
        o_ref[...] = out

    return pl.pallas_call(
        body,
        grid=(N // RB,),
        in_specs=[
            pl.BlockSpec((RB, D), lambda i: (i, 0)),
            pl.BlockSpec((RB, D), lambda i: (i, 0)),
            pl.BlockSpec((RB, CW), lambda i: (i, 0)),
            pl.BlockSpec((RB, CW), lambda i: (i, 0)),
            pl.BlockSpec((RB, D), lambda i: (i, 0)),
            pl.BlockSpec((D, D), lambda i: (0, 0)),
            pl.BlockSpec((1, D), lambda i: (0, 0)),
            pl.BlockSpec((D, D), lambda i: (0, 0)),
        ],
        out_specs=pl.BlockSpec((RB, D), lambda i: (i, 0)),
        out_shape=jax.ShapeDtypeStruct((N, D), jnp.float32),
    )(s0, s1, c0, c1, x, WlT, bl2d, WrT)


def kernel(x, edge_index, Wl1, bl1, Wr1, Wl2, bl2, Wr2):
    src = edge_index[0].astype(jnp.int32)
    dst = edge_index[1].astype(jnp.int32)
    cnts = _sc_cnt(dst)
    c0, c1 = cnts[:N], cnts[NP:NP + N]
    sums1 = _sc_agg(x, src, dst)
    h = _dense(sums1[:N], sums1[NP:NP + N], c0, c1, x, Wl1.T,
               bl1.reshape(1, D), Wr1.T, relu=True)
    sums2 = _sc_agg(h, src, dst)
    out = _dense(sums2[:N], sums2[NP:NP + N], c0, c1, h, Wl2.T,
                 bl2.reshape(1, D), Wr2.T, relu=False)
    return out
